# baseline (speedup 1.0000x reference)
import jax
import jax.numpy as jnp
from jax.experimental import pallas as pl
from jax.experimental.pallas import tpu as pltpu
from jax.experimental.pallas import tpu_sc as plsc

B = 1024
D_TEXT = 768
D_JOINT = 1536
K_DB = 100000
D_PROJ = 384
TOP_K = 3

KEY_TILE = 5000
NUM_TILES = K_DB // KEY_TILE

GATHER_WINDOW = 128


def _mlp_kernel(text_ref, img_ref, gt_ref, w1_ref, b1_ref, w2_ref, b2_ref,
                w3_ref, b3_ref, proj_ref, pn_ref, loss_ref):
    f32 = jnp.float32
    h = jax.lax.dot_general(text_ref[...], w1_ref[:D_TEXT, :],
                            (((1,), (0,)), ((), ())), preferred_element_type=f32)
    h = h + jax.lax.dot_general(img_ref[...], w1_ref[D_TEXT:, :],
                                (((1,), (0,)), ((), ())), preferred_element_type=f32)
    h = jnp.maximum(h + b1_ref[...], 0.0)
    h = jax.lax.dot_general(h, w2_ref[...], (((1,), (0,)), ((), ())),
                            preferred_element_type=f32)
    h = jnp.maximum(h + b2_ref[...], 0.0)
    proj = jax.lax.dot_general(h, w3_ref[...], (((1,), (0,)), ((), ())),
                               preferred_element_type=f32) + b3_ref[...]
    proj_ref[...] = proj
    norm = jnp.sqrt(jnp.sum(proj * proj, axis=1, keepdims=True))
    pn_ref[...] = proj / jnp.clip(norm, 1e-12, None)
    gt = gt_ref[...]
    num = jnp.sum(proj * gt, axis=1, keepdims=True)
    gt_norm = jnp.sqrt(jnp.sum(gt * gt, axis=1, keepdims=True))
    den = jnp.clip(norm, 1e-8, None) * jnp.clip(gt_norm, 1e-8, None)
    loss_ref[...] = jnp.mean(1.0 - num / den, axis=0, keepdims=True)


def _topk_kernel(pn_ref, keys_ref, st_ref):
    t = pl.program_id(0)

    @pl.when(t == 0)
    def _():
        st_ref[...] = jnp.full((8, B), -jnp.inf, jnp.float32)

    s = jax.lax.dot_general(keys_ref[...], pn_ref[...],
                            (((1,), (1,)), ((), ())),
                            preferred_element_type=jnp.float32)
    row = jax.lax.broadcasted_iota(jnp.int32, s.shape, 0)
    base = (t * KEY_TILE).astype(jnp.float32)
    tile_v, tile_i = [], []
    for j in range(TOP_K):
        m = jnp.max(s, axis=0, keepdims=True)
        a = jnp.argmax(s, axis=0, keepdims=True).astype(jnp.int32)
        tile_v.append(m)
        tile_i.append(a.astype(jnp.float32) + base)
        if j < TOP_K - 1:
            s = jnp.where(row == a, -jnp.inf, s)

    v1, v2, v3 = st_ref[0:1, :], st_ref[1:2, :], st_ref[2:3, :]
    i1, i2, i3 = st_ref[3:4, :], st_ref[4:5, :], st_ref[5:6, :]
    m1, m2, m3 = tile_v
    a1, a2, a3 = tile_i

    def vmax(xv, xi, yv, yi):
        c = (xv > yv) | ((xv == yv) & (xi < yi))
        return jnp.where(c, xv, yv), jnp.where(c, xi, yi)

    def vmin(xv, xi, yv, yi):
        c = (xv > yv) | ((xv == yv) & (xi < yi))
        return jnp.where(c, yv, xv), jnp.where(c, yi, xi)

    o1v, o1i = vmax(v1, i1, m1, a1)
    p1v, p1i = vmin(v1, i1, m1, a1)
    q1v, q1i = vmax(v2, i2, m2, a2)
    o2v, o2i = vmax(p1v, p1i, q1v, q1i)
    u1v, u1i = vmin(v1, i1, m2, a2)
    u2v, u2i = vmin(v2, i2, m1, a1)
    u3v, u3i = vmax(v3, i3, m3, a3)
    w1v, w1i = vmax(u1v, u1i, u2v, u2i)
    o3v, o3i = vmax(w1v, w1i, u3v, u3i)

    st_ref[0:1, :] = o1v
    st_ref[1:2, :] = o2v
    st_ref[2:3, :] = o3v
    st_ref[3:4, :] = o1i
    st_ref[4:5, :] = o2i
    st_ref[5:6, :] = o3i


def _sc_gather(kb_embs, idx_flat):
    n = idx_flat.shape[1]
    mesh = plsc.VectorSubcoreMesh(core_axis_name="c", subcore_axis_name="s")

    @pl.kernel(out_type=jax.ShapeDtypeStruct((n, D_PROJ), kb_embs.dtype),
               mesh=mesh)
    def gather_kernel(kb_hbm, i_hbm, o_hbm):
        def body(i_vmem, o_vmem):
            pltpu.sync_copy(kb_hbm.at[i_vmem.at[0]], o_vmem)

        pltpu.emit_pipeline(
            body,
            grid=(n // GATHER_WINDOW,),
            in_specs=[pl.BlockSpec((1, GATHER_WINDOW),
                                   index_map=lambda i: (0, i))],
            out_specs=[pl.BlockSpec((GATHER_WINDOW, D_PROJ),
                                    index_map=lambda i: (i, 0))],
            core_axis_name=("c", "s"),
            dimension_semantics=(pltpu.PARALLEL,),
        )(i_hbm, o_hbm)

    return gather_kernel(kb_embs, idx_flat)


def kernel(text_emb, image_emb, gt_retrievals_emb, W1, b1, W2, b2, W3, b3,
           keys, kb_embs):
    f32 = jnp.float32
    proj, pn, loss = pl.pallas_call(
        _mlp_kernel,
        out_shape=(
            jax.ShapeDtypeStruct((B, D_PROJ), f32),
            jax.ShapeDtypeStruct((B, D_PROJ), f32),
            jax.ShapeDtypeStruct((1, 1), f32),
        ),
    )(text_emb, image_emb, gt_retrievals_emb, W1, b1.reshape(1, -1),
      W2, b2.reshape(1, -1), W3, b3.reshape(1, -1))

    st = pl.pallas_call(
        _topk_kernel,
        grid=(NUM_TILES,),
        in_specs=[
            pl.BlockSpec((B, D_PROJ), lambda t: (0, 0)),
            pl.BlockSpec((KEY_TILE, D_PROJ), lambda t: (t, 0)),
        ],
        out_specs=pl.BlockSpec((8, B), lambda t: (0, 0)),
        out_shape=jax.ShapeDtypeStruct((8, B), f32),
        compiler_params=pltpu.CompilerParams(
            dimension_semantics=("arbitrary",)),
    )(pn, keys)

    idx = st[3:6, :].astype(jnp.int32)
    idx_flat = idx.T.reshape(1, B * TOP_K)
    retrieved = _sc_gather(kb_embs, idx_flat)
    return retrieved.reshape(B, TOP_K, D_PROJ), proj, loss.reshape(())


# speedup vs baseline: 1.0014x; 1.0014x over previous
import jax
import jax.numpy as jnp
from jax.experimental import pallas as pl
from jax.experimental.pallas import tpu as pltpu
from jax.experimental.pallas import tpu_sc as plsc

B = 1024
D_TEXT = 768
D_JOINT = 1536
K_DB = 100000
D_PROJ = 384
TOP_K = 3

KEY_TILE = 4000
NUM_TILES = K_DB // KEY_TILE

GATHER_WINDOW = 128


def _mlp_kernel(text_ref, img_ref, gt_ref, w1_ref, b1_ref, w2_ref, b2_ref,
                w3_ref, b3_ref, proj_ref, pn_ref, loss_ref):
    f32 = jnp.float32
    h = jax.lax.dot_general(text_ref[...], w1_ref[:D_TEXT, :],
                            (((1,), (0,)), ((), ())), preferred_element_type=f32)
    h = h + jax.lax.dot_general(img_ref[...], w1_ref[D_TEXT:, :],
                                (((1,), (0,)), ((), ())), preferred_element_type=f32)
    h = jnp.maximum(h + b1_ref[...], 0.0)
    h = jax.lax.dot_general(h, w2_ref[...], (((1,), (0,)), ((), ())),
                            preferred_element_type=f32)
    h = jnp.maximum(h + b2_ref[...], 0.0)
    proj = jax.lax.dot_general(h, w3_ref[...], (((1,), (0,)), ((), ())),
                               preferred_element_type=f32) + b3_ref[...]
    proj_ref[...] = proj
    norm = jnp.sqrt(jnp.sum(proj * proj, axis=1, keepdims=True))
    pn_ref[...] = proj / jnp.clip(norm, 1e-12, None)
    gt = gt_ref[...]
    num = jnp.sum(proj * gt, axis=1, keepdims=True)
    gt_norm = jnp.sqrt(jnp.sum(gt * gt, axis=1, keepdims=True))
    den = jnp.clip(norm, 1e-8, None) * jnp.clip(gt_norm, 1e-8, None)
    loss_ref[...] = jnp.mean(1.0 - num / den, axis=0, keepdims=True)


def _topk_kernel(pn_ref, keys_ref, st_ref):
    t = pl.program_id(0)

    @pl.when(t == 0)
    def _():
        st_ref[...] = jnp.full((8, B), -jnp.inf, jnp.float32)

    s = jax.lax.dot_general(keys_ref[...], pn_ref[...],
                            (((1,), (1,)), ((), ())),
                            preferred_element_type=jnp.float32)
    row = jax.lax.broadcasted_iota(jnp.int32, s.shape, 0)
    base = (t * KEY_TILE).astype(jnp.float32)
    tile_v, tile_i = [], []
    for j in range(TOP_K):
        m = jnp.max(s, axis=0, keepdims=True)
        a = jnp.argmax(s, axis=0, keepdims=True).astype(jnp.int32)
        tile_v.append(m)
        tile_i.append(a.astype(jnp.float32) + base)
        if j < TOP_K - 1:
            s = jnp.where(row == a, -jnp.inf, s)

    v1, v2, v3 = st_ref[0:1, :], st_ref[1:2, :], st_ref[2:3, :]
    i1, i2, i3 = st_ref[3:4, :], st_ref[4:5, :], st_ref[5:6, :]
    m1, m2, m3 = tile_v
    a1, a2, a3 = tile_i

    def vmax(xv, xi, yv, yi):
        c = (xv > yv) | ((xv == yv) & (xi < yi))
        return jnp.where(c, xv, yv), jnp.where(c, xi, yi)

    def vmin(xv, xi, yv, yi):
        c = (xv > yv) | ((xv == yv) & (xi < yi))
        return jnp.where(c, yv, xv), jnp.where(c, yi, xi)

    o1v, o1i = vmax(v1, i1, m1, a1)
    p1v, p1i = vmin(v1, i1, m1, a1)
    q1v, q1i = vmax(v2, i2, m2, a2)
    o2v, o2i = vmax(p1v, p1i, q1v, q1i)
    u1v, u1i = vmin(v1, i1, m2, a2)
    u2v, u2i = vmin(v2, i2, m1, a1)
    u3v, u3i = vmax(v3, i3, m3, a3)
    w1v, w1i = vmax(u1v, u1i, u2v, u2i)
    o3v, o3i = vmax(w1v, w1i, u3v, u3i)

    st_ref[0:1, :] = o1v
    st_ref[1:2, :] = o2v
    st_ref[2:3, :] = o3v
    st_ref[3:4, :] = o1i
    st_ref[4:5, :] = o2i
    st_ref[5:6, :] = o3i


def _sc_gather(kb_embs, idx_flat):
    n = idx_flat.shape[1]
    mesh = plsc.VectorSubcoreMesh(core_axis_name="c", subcore_axis_name="s")

    @pl.kernel(out_type=jax.ShapeDtypeStruct((n, D_PROJ), kb_embs.dtype),
               mesh=mesh)
    def gather_kernel(kb_hbm, i_hbm, o_hbm):
        def body(i_vmem, o_vmem):
            pltpu.sync_copy(kb_hbm.at[i_vmem.at[0]], o_vmem)

        pltpu.emit_pipeline(
            body,
            grid=(n // GATHER_WINDOW,),
            in_specs=[pl.BlockSpec((1, GATHER_WINDOW),
                                   index_map=lambda i: (0, i))],
            out_specs=[pl.BlockSpec((GATHER_WINDOW, D_PROJ),
                                    index_map=lambda i: (i, 0))],
            core_axis_name=("c", "s"),
            dimension_semantics=(pltpu.PARALLEL,),
        )(i_hbm, o_hbm)

    return gather_kernel(kb_embs, idx_flat)


def kernel(text_emb, image_emb, gt_retrievals_emb, W1, b1, W2, b2, W3, b3,
           keys, kb_embs):
    f32 = jnp.float32
    proj, pn, loss = pl.pallas_call(
        _mlp_kernel,
        out_shape=(
            jax.ShapeDtypeStruct((B, D_PROJ), f32),
            jax.ShapeDtypeStruct((B, D_PROJ), f32),
            jax.ShapeDtypeStruct((1, 1), f32),
        ),
    )(text_emb, image_emb, gt_retrievals_emb, W1, b1.reshape(1, -1),
      W2, b2.reshape(1, -1), W3, b3.reshape(1, -1))

    st = pl.pallas_call(
        _topk_kernel,
        grid=(NUM_TILES,),
        in_specs=[
            pl.BlockSpec((B, D_PROJ), lambda t: (0, 0)),
            pl.BlockSpec((KEY_TILE, D_PROJ), lambda t: (t, 0)),
        ],
        out_specs=pl.BlockSpec((8, B), lambda t: (0, 0)),
        out_shape=jax.ShapeDtypeStruct((8, B), f32),
        compiler_params=pltpu.CompilerParams(
            dimension_semantics=("arbitrary",)),
    )(pn, keys)

    idx = st[3:6, :].astype(jnp.int32)
    idx_flat = idx.T.reshape(1, B * TOP_K)
    retrieved = _sc_gather(kb_embs, idx_flat)
    return retrieved.reshape(B, TOP_K, D_PROJ), proj, loss.reshape(())
